# Initial kernel scaffold; baseline (speedup 1.0000x reference)
import jax
import jax.numpy as jnp
from jax.experimental import pallas as pl


def kernel(table, attention_mask, table_labels_S, table_labels_E, aspect_pred_tags, opinion_pred_tags, aspect_golde_tags, opinion_golde_tags, biaffine_edge_S, biaffine_edge_E, W_S, b_S, W_E, b_E):
    raise NotImplementedError("write your pallas kernel here")


# speedup vs baseline: 1.6295x; 1.6295x over previous
import jax
import jax.numpy as jnp
from jax.experimental import pallas as pl
from jax.experimental.pallas import tpu as pltpu


def _matvec_kernel(x_ref, w_ref, b_ref, o_ref):
    o_ref[...] = (
        jnp.dot(x_ref[...], w_ref[...], preferred_element_type=jnp.float32)
        + b_ref[...]
    )


def _stage2_kernel(am_ref, ltS_ref, ltE_ref, biaS_ref, biaE_ref,
                   labS_ref, labE_ref, ap_ref, op_ref, ag_ref, og_ref,
                   mS_ref, mE_ref, lS_ref, lE_ref, la_ref, lo_ref, acc_ref):
    b = pl.program_id(0)
    nb = pl.num_programs(0)

    @pl.when(b == 0)
    def _init():
        for i in range(6):
            acc_ref[i] = jnp.float32(0.0)

    labS = labS_ref[0]
    labE = labE_ref[0]
    w = (labS >= 0).astype(jnp.float32)
    xS = ltS_ref[0] * (1.0 + biaS_ref[0])
    xE = ltE_ref[0] * (1.0 + biaE_ref[0])
    yS = labS.astype(jnp.float32)
    yE = labE.astype(jnp.float32)
    elemS = jnp.maximum(xS, 0.0) - xS * yS + jnp.log1p(jnp.exp(-jnp.abs(xS)))
    elemE = jnp.maximum(xE, 0.0) - xE * yE + jnp.log1p(jnp.exp(-jnp.abs(xE)))
    acc_ref[0] += jnp.sum(w * elemS)
    acc_ref[1] += jnp.sum(w * elemE)

    def ce_partial(x, tgt):
        m = jnp.max(x, axis=0, keepdims=True)
        lse = m + jnp.log(jnp.sum(jnp.exp(x - m), axis=0, keepdims=True))
        valid = tgt != -1
        vf = valid.astype(jnp.float32)
        st = jnp.where(valid, tgt, 0)
        oh = (jax.lax.broadcasted_iota(jnp.int32, x.shape, 0) == st).astype(
            jnp.float32)
        xt = jnp.sum(x * oh, axis=0, keepdims=True)
        return jnp.sum((lse - xt) * vf), jnp.sum(vf)

    na, ca = ce_partial(ap_ref[0], ag_ref[pl.ds(b, 1), :])
    no, co = ce_partial(op_ref[0], og_ref[pl.ds(b, 1), :])
    acc_ref[2] += na
    acc_ref[3] += ca
    acc_ref[4] += no
    acc_ref[5] += co

    amrow = am_ref[pl.ds(b, 1), :]
    mask_len = jnp.sum(amrow) - 2
    length = (mask_len.astype(jnp.float32) * 0.3).astype(jnp.int32)
    length = jnp.maximum(length, 5)
    length = jnp.minimum(length, mask_len * mask_len)

    def topk_mask(pred):
        bits = jax.lax.bitcast_convert_type(pred, jnp.int32)

        def body(_, lh):
            lo, hi = lh
            mid = jax.lax.div(lo + hi, jnp.int32(2))
            cnt = jnp.sum((bits >= mid).astype(jnp.int32))
            ok = cnt >= length
            return jnp.where(ok, mid, lo), jnp.where(ok, hi, mid)

        lo, _ = jax.lax.fori_loop(
            0, 31, body, (jnp.int32(0), jnp.int32(0x3F800001)))
        return (bits >= lo).astype(jnp.int32)

    mS_ref[0] = topk_mask(jax.nn.sigmoid(xS) * w)
    mE_ref[0] = topk_mask(jax.nn.sigmoid(xE) * w)

    @pl.when(b == nb - 1)
    def _final():
        nelem = jnp.float32(nb * xS.shape[0] * xS.shape[1])
        lS_ref[...] = jnp.reshape(acc_ref[0] / nelem, (1, 1))
        lE_ref[...] = jnp.reshape(acc_ref[1] / nelem, (1, 1))
        la_ref[...] = jnp.reshape(
            0.1 * acc_ref[2] / jnp.maximum(acc_ref[3], 1.0), (1, 1))
        lo_ref[...] = jnp.reshape(
            0.1 * acc_ref[4] / jnp.maximum(acc_ref[5], 1.0), (1, 1))


def kernel(table, attention_mask, table_labels_S, table_labels_E,
           aspect_pred_tags, opinion_pred_tags, aspect_golde_tags,
           opinion_golde_tags, biaffine_edge_S, biaffine_edge_E,
           W_S, b_S, W_E, b_E):
    B, Lq, Lk, D = table.shape
    M = B * Lq * Lk
    x = table.reshape(M, D)
    W2 = jnp.concatenate([W_S, W_E], axis=1)
    b2 = jnp.concatenate([b_S, b_E]).reshape(1, 2)

    BLK = 2048
    lt = pl.pallas_call(
        _matvec_kernel,
        grid=(M // BLK,),
        in_specs=[
            pl.BlockSpec((BLK, D), lambda i: (i, 0)),
            pl.BlockSpec((D, 2), lambda i: (0, 0)),
            pl.BlockSpec((1, 2), lambda i: (0, 0)),
        ],
        out_specs=pl.BlockSpec((BLK, 2), lambda i: (i, 0)),
        out_shape=jax.ShapeDtypeStruct((M, 2), jnp.float32),
    )(x, W2, b2)

    ltS = lt[:, 0].reshape(B, Lq, Lk)
    ltE = lt[:, 1].reshape(B, Lq, Lk)
    biaS = jnp.squeeze(biaffine_edge_S, 3)
    biaE = jnp.squeeze(biaffine_edge_E, 3)
    ap_t = jnp.transpose(aspect_pred_tags, (0, 2, 1))
    op_t = jnp.transpose(opinion_pred_tags, (0, 2, 1))
    C = ap_t.shape[1]

    full2 = lambda shape: pl.BlockSpec(shape, lambda b: (0, 0))
    per_b = lambda s1, s2: pl.BlockSpec((1, s1, s2), lambda b: (b, 0, 0))

    outs = pl.pallas_call(
        _stage2_kernel,
        grid=(B,),
        in_specs=[
            full2((B, Lq)),
            per_b(Lq, Lk),
            per_b(Lq, Lk),
            per_b(Lq, Lk),
            per_b(Lq, Lk),
            per_b(Lq, Lk),
            per_b(Lq, Lk),
            per_b(C, Lq),
            per_b(C, Lq),
            full2((B, Lq)),
            full2((B, Lq)),
        ],
        out_specs=[
            per_b(Lq, Lk),
            per_b(Lq, Lk),
            full2((1, 1)),
            full2((1, 1)),
            full2((1, 1)),
            full2((1, 1)),
        ],
        out_shape=[
            jax.ShapeDtypeStruct((B, Lq, Lk), jnp.int32),
            jax.ShapeDtypeStruct((B, Lq, Lk), jnp.int32),
            jax.ShapeDtypeStruct((1, 1), jnp.float32),
            jax.ShapeDtypeStruct((1, 1), jnp.float32),
            jax.ShapeDtypeStruct((1, 1), jnp.float32),
            jax.ShapeDtypeStruct((1, 1), jnp.float32),
        ],
        scratch_shapes=[pltpu.SMEM((8,), jnp.float32)],
    )(attention_mask, ltS, ltE, biaS, biaE, table_labels_S, table_labels_E,
      ap_t, op_t, aspect_golde_tags, opinion_golde_tags)

    mS, mE, lS, lE, la, lo = outs
    return (lS.reshape(()), lE.reshape(()), la.reshape(()), lo.reshape(()),
            mS.astype(jnp.bool_), mE.astype(jnp.bool_))


# speedup vs baseline: 2.0578x; 1.2629x over previous
import jax
import jax.numpy as jnp
from jax.experimental import pallas as pl
from jax.experimental.pallas import tpu as pltpu


def _matvec_kernel(x_ref, w_ref, b_ref, o_ref):
    o_ref[...] = (
        jnp.dot(x_ref[...], w_ref[...], preferred_element_type=jnp.float32)
        + b_ref[...]
    )


def _stage2_kernel(am_ref, ltS_ref, ltE_ref, biaS_ref, biaE_ref,
                   labS_ref, labE_ref, ap_ref, op_ref, ag_ref, og_ref,
                   mS_ref, mE_ref, lS_ref, lE_ref, la_ref, lo_ref):
    labS = labS_ref[...]
    labE = labE_ref[...]
    w = (labS >= 0).astype(jnp.float32)
    xS = ltS_ref[...] * (1.0 + biaS_ref[...])
    xE = ltE_ref[...] * (1.0 + biaE_ref[...])
    yS = labS.astype(jnp.float32)
    yE = labE.astype(jnp.float32)
    elemS = jnp.maximum(xS, 0.0) - xS * yS + jnp.log1p(jnp.exp(-jnp.abs(xS)))
    elemE = jnp.maximum(xE, 0.0) - xE * yE + jnp.log1p(jnp.exp(-jnp.abs(xE)))
    nelem = jnp.float32(xS.size)
    lS_ref[...] = jnp.reshape(jnp.sum(w * elemS) / nelem, (1, 1))
    lE_ref[...] = jnp.reshape(jnp.sum(w * elemE) / nelem, (1, 1))

    def ce_loss(x, tgt):
        m = jnp.max(x, axis=1, keepdims=True)
        lse = m + jnp.log(jnp.sum(jnp.exp(x - m), axis=1, keepdims=True))
        valid = tgt != -1
        vf = valid.astype(jnp.float32)
        st = jnp.where(valid, tgt, 0)
        oh = (jax.lax.broadcasted_iota(jnp.int32, x.shape, 1) == st).astype(
            jnp.float32)
        xt = jnp.sum(x * oh, axis=1, keepdims=True)
        nll = jnp.sum((lse - xt) * vf)
        return 0.1 * nll / jnp.maximum(jnp.sum(vf), 1.0)

    la_ref[...] = jnp.reshape(ce_loss(ap_ref[...], ag_ref[...]), (1, 1))
    lo_ref[...] = jnp.reshape(ce_loss(op_ref[...], og_ref[...]), (1, 1))

    mask_len = jnp.sum(am_ref[...], axis=2, keepdims=True) - 2
    length = (mask_len.astype(jnp.float32) * 0.3).astype(jnp.int32)
    length = jnp.maximum(length, 5)
    length = jnp.minimum(length, mask_len * mask_len)

    bitsS = jax.lax.bitcast_convert_type(jax.nn.sigmoid(xS) * w, jnp.int32)
    bitsE = jax.lax.bitcast_convert_type(jax.nn.sigmoid(xE) * w, jnp.int32)

    def body(_, carry):
        loS, hiS, loE, hiE = carry
        midS = jax.lax.div(loS + hiS, jnp.int32(2))
        midE = jax.lax.div(loE + hiE, jnp.int32(2))
        cntS = jnp.sum((bitsS >= midS).astype(jnp.int32), axis=(1, 2),
                       keepdims=True)
        cntE = jnp.sum((bitsE >= midE).astype(jnp.int32), axis=(1, 2),
                       keepdims=True)
        okS = cntS >= length
        okE = cntE >= length
        return (jnp.where(okS, midS, loS), jnp.where(okS, hiS, midS),
                jnp.where(okE, midE, loE), jnp.where(okE, hiE, midE))

    B = xS.shape[0]
    zeros = jnp.zeros((B, 1, 1), jnp.int32)
    ones = jnp.full((B, 1, 1), 0x3F800001, jnp.int32)
    loS, _, loE, _ = jax.lax.fori_loop(0, 31, body,
                                       (zeros, ones, zeros, ones))
    mS_ref[...] = (bitsS >= loS).astype(jnp.int32)
    mE_ref[...] = (bitsE >= loE).astype(jnp.int32)


def kernel(table, attention_mask, table_labels_S, table_labels_E,
           aspect_pred_tags, opinion_pred_tags, aspect_golde_tags,
           opinion_golde_tags, biaffine_edge_S, biaffine_edge_E,
           W_S, b_S, W_E, b_E):
    B, Lq, Lk, D = table.shape
    M = B * Lq * Lk
    x = table.reshape(M, D)
    W2 = jnp.concatenate([W_S, W_E], axis=1)
    b2 = jnp.concatenate([b_S, b_E]).reshape(1, 2)

    BLK = 2048
    lt = pl.pallas_call(
        _matvec_kernel,
        grid=(M // BLK,),
        in_specs=[
            pl.BlockSpec((BLK, D), lambda i: (i, 0)),
            pl.BlockSpec((D, 2), lambda i: (0, 0)),
            pl.BlockSpec((1, 2), lambda i: (0, 0)),
        ],
        out_specs=pl.BlockSpec((BLK, 2), lambda i: (i, 0)),
        out_shape=jax.ShapeDtypeStruct((M, 2), jnp.float32),
    )(x, W2, b2)

    ltS = lt[:, 0].reshape(B, Lq, Lk)
    ltE = lt[:, 1].reshape(B, Lq, Lk)
    biaS = jnp.squeeze(biaffine_edge_S, 3)
    biaE = jnp.squeeze(biaffine_edge_E, 3)
    ap_t = jnp.transpose(aspect_pred_tags, (0, 2, 1))
    op_t = jnp.transpose(opinion_pred_tags, (0, 2, 1))
    am3 = attention_mask.reshape(B, 1, Lq)
    ag3 = aspect_golde_tags.reshape(B, 1, Lq)
    og3 = opinion_golde_tags.reshape(B, 1, Lq)

    outs = pl.pallas_call(
        _stage2_kernel,
        out_shape=[
            jax.ShapeDtypeStruct((B, Lq, Lk), jnp.int32),
            jax.ShapeDtypeStruct((B, Lq, Lk), jnp.int32),
            jax.ShapeDtypeStruct((1, 1), jnp.float32),
            jax.ShapeDtypeStruct((1, 1), jnp.float32),
            jax.ShapeDtypeStruct((1, 1), jnp.float32),
            jax.ShapeDtypeStruct((1, 1), jnp.float32),
        ],
    )(am3, ltS, ltE, biaS, biaE, table_labels_S, table_labels_E,
      ap_t, op_t, ag3, og3)

    mS, mE, lS, lE, la, lo = outs
    return (lS.reshape(()), lE.reshape(()), la.reshape(()), lo.reshape(()),
            mS.astype(jnp.bool_), mE.astype(jnp.bool_))


# speedup vs baseline: 2.3746x; 1.1539x over previous
import jax
import jax.numpy as jnp
from jax.experimental import pallas as pl
from jax.experimental.pallas import tpu as pltpu


def _fused_kernel(x_ref, w_ref, bt_ref, biaT_ref, labT_ref, am_ref,
                  ap_ref, op_ref, ag_ref, og_ref,
                  m_ref, lS_ref, lE_ref, la_ref, lo_ref,
                  bits_ref, acc_ref):
    b = pl.program_id(0)
    m = pl.program_id(1)
    nb = pl.num_programs(0)
    nm = pl.num_programs(1)
    blk = x_ref.shape[1]

    @pl.when((b == 0) & (m == 0))
    def _init():
        for i in range(6):
            acc_ref[i] = jnp.float32(0.0)

    lt2 = jnp.dot(x_ref[0], w_ref[...], preferred_element_type=jnp.float32)
    ltT = jnp.transpose(lt2)
    xsc = (ltT + bt_ref[...]) * (1.0 + biaT_ref[0])
    labT = labT_ref[0]
    w = (labT[0:1, :] >= 0).astype(jnp.float32)
    y = labT.astype(jnp.float32)
    elem = jnp.maximum(xsc, 0.0) - xsc * y + jnp.log1p(jnp.exp(-jnp.abs(xsc)))
    welem = w * elem
    acc_ref[0] += jnp.sum(welem[0:1, :])
    acc_ref[1] += jnp.sum(welem[1:2, :])
    pred = jax.nn.sigmoid(xsc) * w
    bits_ref[:, pl.ds(m * blk, blk)] = jax.lax.bitcast_convert_type(
        pred, jnp.int32)

    @pl.when(m == 0)
    def _ce():
        def ce_acc(x, tgt, i0):
            mx = jnp.max(x, axis=0, keepdims=True)
            lse = mx + jnp.log(jnp.sum(jnp.exp(x - mx), axis=0, keepdims=True))
            valid = tgt != -1
            vf = valid.astype(jnp.float32)
            st = jnp.where(valid, tgt, 0)
            oh = (jax.lax.broadcasted_iota(jnp.int32, x.shape, 0) == st
                  ).astype(jnp.float32)
            xt = jnp.sum(x * oh, axis=0, keepdims=True)
            acc_ref[i0] += jnp.sum((lse - xt) * vf)
            acc_ref[i0 + 1] += jnp.sum(vf)

        ce_acc(ap_ref[0], ag_ref[0], 2)
        ce_acc(op_ref[0], og_ref[0], 4)

    @pl.when(m == nm - 1)
    def _prune():
        mask_len = jnp.sum(am_ref[0]) - 2
        length = (mask_len.astype(jnp.float32) * 0.3).astype(jnp.int32)
        length = jnp.maximum(length, 5)
        length = jnp.minimum(length, mask_len * mask_len)
        bits = bits_ref[...]

        def body(_, lh):
            lo, hi = lh
            mid = jax.lax.div(lo + hi, jnp.int32(2))
            cnt = jnp.sum((bits >= mid).astype(jnp.int32), axis=1,
                          keepdims=True)
            ok = cnt >= length
            return jnp.where(ok, mid, lo), jnp.where(ok, hi, mid)

        lo, _ = jax.lax.fori_loop(
            0, 31, body,
            (jnp.zeros((2, 1), jnp.int32),
             jnp.full((2, 1), 0x3F800001, jnp.int32)))
        m_ref[0] = (bits >= lo).astype(jnp.int32)

    @pl.when((b == nb - 1) & (m == nm - 1))
    def _final():
        nelem = jnp.float32(nb * nm * blk)
        lS_ref[...] = jnp.reshape(acc_ref[0] / nelem, (1, 1))
        lE_ref[...] = jnp.reshape(acc_ref[1] / nelem, (1, 1))
        la_ref[...] = jnp.reshape(
            0.1 * acc_ref[2] / jnp.maximum(acc_ref[3], 1.0), (1, 1))
        lo_ref[...] = jnp.reshape(
            0.1 * acc_ref[4] / jnp.maximum(acc_ref[5], 1.0), (1, 1))


def kernel(table, attention_mask, table_labels_S, table_labels_E,
           aspect_pred_tags, opinion_pred_tags, aspect_golde_tags,
           opinion_golde_tags, biaffine_edge_S, biaffine_edge_E,
           W_S, b_S, W_E, b_E):
    B, Lq, Lk, D = table.shape
    LL = Lq * Lk
    BLK = 2048
    NM = LL // BLK

    x = table.reshape(B, LL, D)
    W2 = jnp.concatenate([W_S, W_E], axis=1)
    b2T = jnp.concatenate([b_S, b_E]).reshape(2, 1)
    biaT = jnp.stack([biaffine_edge_S.reshape(B, LL),
                      biaffine_edge_E.reshape(B, LL)], axis=1)
    labT = jnp.stack([table_labels_S.reshape(B, LL),
                      table_labels_E.reshape(B, LL)], axis=1)
    am3 = attention_mask.reshape(B, 1, Lq)
    ap_t = jnp.transpose(aspect_pred_tags, (0, 2, 1))
    op_t = jnp.transpose(opinion_pred_tags, (0, 2, 1))
    ag3 = aspect_golde_tags.reshape(B, 1, Lq)
    og3 = opinion_golde_tags.reshape(B, 1, Lq)
    C = ap_t.shape[1]

    outs = pl.pallas_call(
        _fused_kernel,
        grid=(B, NM),
        in_specs=[
            pl.BlockSpec((1, BLK, D), lambda b, m: (b, m, 0)),
            pl.BlockSpec((D, 2), lambda b, m: (0, 0)),
            pl.BlockSpec((2, 1), lambda b, m: (0, 0)),
            pl.BlockSpec((1, 2, BLK), lambda b, m: (b, 0, m)),
            pl.BlockSpec((1, 2, BLK), lambda b, m: (b, 0, m)),
            pl.BlockSpec((1, 1, Lq), lambda b, m: (b, 0, 0)),
            pl.BlockSpec((1, C, Lq), lambda b, m: (b, 0, 0)),
            pl.BlockSpec((1, C, Lq), lambda b, m: (b, 0, 0)),
            pl.BlockSpec((1, 1, Lq), lambda b, m: (b, 0, 0)),
            pl.BlockSpec((1, 1, Lq), lambda b, m: (b, 0, 0)),
        ],
        out_specs=[
            pl.BlockSpec((1, 2, LL), lambda b, m: (b, 0, 0)),
            pl.BlockSpec((1, 1), lambda b, m: (0, 0)),
            pl.BlockSpec((1, 1), lambda b, m: (0, 0)),
            pl.BlockSpec((1, 1), lambda b, m: (0, 0)),
            pl.BlockSpec((1, 1), lambda b, m: (0, 0)),
        ],
        out_shape=[
            jax.ShapeDtypeStruct((B, 2, LL), jnp.int32),
            jax.ShapeDtypeStruct((1, 1), jnp.float32),
            jax.ShapeDtypeStruct((1, 1), jnp.float32),
            jax.ShapeDtypeStruct((1, 1), jnp.float32),
            jax.ShapeDtypeStruct((1, 1), jnp.float32),
        ],
        scratch_shapes=[
            pltpu.VMEM((2, LL), jnp.int32),
            pltpu.SMEM((8,), jnp.float32),
        ],
    )(x, W2, b2T, biaT, labT, am3, ap_t, op_t, ag3, og3)

    masks, lS, lE, la, lo = outs
    mS = masks[:, 0, :].reshape(B, Lq, Lk).astype(jnp.bool_)
    mE = masks[:, 1, :].reshape(B, Lq, Lk).astype(jnp.bool_)
    return (lS.reshape(()), lE.reshape(()), la.reshape(()), lo.reshape(()),
            mS, mE)


# speedup vs baseline: 2.5540x; 1.0756x over previous
import jax
import jax.numpy as jnp
from jax.experimental import pallas as pl
from jax.experimental.pallas import tpu as pltpu


def _fused_kernel(x_ref, w_ref, bt_ref, biaT_ref, labT_ref, am_ref,
                  ap_ref, op_ref, ag_ref, og_ref,
                  m_ref, lS_ref, lE_ref, la_ref, lo_ref,
                  bits_ref, acc_ref):
    b = pl.program_id(0)
    m = pl.program_id(1)
    nb = pl.num_programs(0)
    nm = pl.num_programs(1)
    blk = x_ref.shape[1]

    @pl.when((b == 0) & (m == 0))
    def _init():
        for i in range(6):
            acc_ref[i] = jnp.float32(0.0)

    lt2 = jnp.dot(x_ref[0], w_ref[...], preferred_element_type=jnp.float32)
    ltT = jnp.transpose(lt2)
    xsc = (ltT + bt_ref[...]) * (1.0 + biaT_ref[0])
    labT = labT_ref[0]
    w = (labT[0:1, :] >= 0).astype(jnp.float32)
    y = labT.astype(jnp.float32)
    elem = jnp.maximum(xsc, 0.0) - xsc * y + jnp.log1p(jnp.exp(-jnp.abs(xsc)))
    welem = w * elem
    acc_ref[0] += jnp.sum(welem[0:1, :])
    acc_ref[1] += jnp.sum(welem[1:2, :])
    pred = jax.nn.sigmoid(xsc) * w
    bits_ref[:, pl.ds(m * blk, blk)] = jax.lax.bitcast_convert_type(
        pred, jnp.int32)

    @pl.when(m == 0)
    def _ce():
        def ce_acc(x, tgt, i0):
            mx = jnp.max(x, axis=0, keepdims=True)
            lse = mx + jnp.log(jnp.sum(jnp.exp(x - mx), axis=0, keepdims=True))
            valid = tgt != -1
            vf = valid.astype(jnp.float32)
            st = jnp.where(valid, tgt, 0)
            oh = (jax.lax.broadcasted_iota(jnp.int32, x.shape, 0) == st
                  ).astype(jnp.float32)
            xt = jnp.sum(x * oh, axis=0, keepdims=True)
            acc_ref[i0] += jnp.sum((lse - xt) * vf)
            acc_ref[i0 + 1] += jnp.sum(vf)

        ce_acc(ap_ref[0], ag_ref[0], 2)
        ce_acc(op_ref[0], og_ref[0], 4)

    @pl.when(m == nm - 1)
    def _prune():
        mask_len = jnp.sum(am_ref[0]) - 2
        length = (mask_len.astype(jnp.float32) * 0.3).astype(jnp.int32)
        length = jnp.maximum(length, 5)
        length = jnp.minimum(length, mask_len * mask_len)
        bits = bits_ref[...]

        def body(_, lh):
            lo, hi = lh
            mid = jax.lax.div(lo + hi, jnp.int32(2))
            cnt = jnp.sum((bits >= mid).astype(jnp.int32), axis=1,
                          keepdims=True)
            ok = cnt >= length
            return jnp.where(ok, mid, lo), jnp.where(ok, hi, mid)

        lo, _ = jax.lax.fori_loop(
            0, 31, body,
            (jnp.zeros((2, 1), jnp.int32),
             jnp.full((2, 1), 0x3F800001, jnp.int32)))
        m_ref[0] = (bits >= lo).astype(jnp.int32)

    @pl.when((b == nb - 1) & (m == nm - 1))
    def _final():
        nelem = jnp.float32(nb * nm * blk)
        lS_ref[...] = jnp.reshape(acc_ref[0] / nelem, (1, 1))
        lE_ref[...] = jnp.reshape(acc_ref[1] / nelem, (1, 1))
        la_ref[...] = jnp.reshape(
            0.1 * acc_ref[2] / jnp.maximum(acc_ref[3], 1.0), (1, 1))
        lo_ref[...] = jnp.reshape(
            0.1 * acc_ref[4] / jnp.maximum(acc_ref[5], 1.0), (1, 1))


def kernel(table, attention_mask, table_labels_S, table_labels_E,
           aspect_pred_tags, opinion_pred_tags, aspect_golde_tags,
           opinion_golde_tags, biaffine_edge_S, biaffine_edge_E,
           W_S, b_S, W_E, b_E):
    B, Lq, Lk, D = table.shape
    LL = Lq * Lk
    BLK = 4096
    NM = LL // BLK

    x = table.reshape(B, LL, D)
    W2 = jnp.concatenate([W_S, W_E], axis=1)
    b2T = jnp.concatenate([b_S, b_E]).reshape(2, 1)
    biaT = jnp.stack([biaffine_edge_S.reshape(B, LL),
                      biaffine_edge_E.reshape(B, LL)], axis=1)
    labT = jnp.stack([table_labels_S.reshape(B, LL),
                      table_labels_E.reshape(B, LL)], axis=1)
    am3 = attention_mask.reshape(B, 1, Lq)
    ap_t = jnp.transpose(aspect_pred_tags, (0, 2, 1))
    op_t = jnp.transpose(opinion_pred_tags, (0, 2, 1))
    ag3 = aspect_golde_tags.reshape(B, 1, Lq)
    og3 = opinion_golde_tags.reshape(B, 1, Lq)
    C = ap_t.shape[1]

    outs = pl.pallas_call(
        _fused_kernel,
        grid=(B, NM),
        in_specs=[
            pl.BlockSpec((1, BLK, D), lambda b, m: (b, m, 0)),
            pl.BlockSpec((D, 2), lambda b, m: (0, 0)),
            pl.BlockSpec((2, 1), lambda b, m: (0, 0)),
            pl.BlockSpec((1, 2, BLK), lambda b, m: (b, 0, m)),
            pl.BlockSpec((1, 2, BLK), lambda b, m: (b, 0, m)),
            pl.BlockSpec((1, 1, Lq), lambda b, m: (b, 0, 0)),
            pl.BlockSpec((1, C, Lq), lambda b, m: (b, 0, 0)),
            pl.BlockSpec((1, C, Lq), lambda b, m: (b, 0, 0)),
            pl.BlockSpec((1, 1, Lq), lambda b, m: (b, 0, 0)),
            pl.BlockSpec((1, 1, Lq), lambda b, m: (b, 0, 0)),
        ],
        out_specs=[
            pl.BlockSpec((1, 2, LL), lambda b, m: (b, 0, 0)),
            pl.BlockSpec((1, 1), lambda b, m: (0, 0)),
            pl.BlockSpec((1, 1), lambda b, m: (0, 0)),
            pl.BlockSpec((1, 1), lambda b, m: (0, 0)),
            pl.BlockSpec((1, 1), lambda b, m: (0, 0)),
        ],
        out_shape=[
            jax.ShapeDtypeStruct((B, 2, LL), jnp.int32),
            jax.ShapeDtypeStruct((1, 1), jnp.float32),
            jax.ShapeDtypeStruct((1, 1), jnp.float32),
            jax.ShapeDtypeStruct((1, 1), jnp.float32),
            jax.ShapeDtypeStruct((1, 1), jnp.float32),
        ],
        scratch_shapes=[
            pltpu.VMEM((2, LL), jnp.int32),
            pltpu.SMEM((8,), jnp.float32),
        ],
    )(x, W2, b2T, biaT, labT, am3, ap_t, op_t, ag3, og3)

    masks, lS, lE, la, lo = outs
    mS = masks[:, 0, :].reshape(B, Lq, Lk).astype(jnp.bool_)
    mE = masks[:, 1, :].reshape(B, Lq, Lk).astype(jnp.bool_)
    return (lS.reshape(()), lE.reshape(()), la.reshape(()), lo.reshape(()),
            mS, mE)


# speedup vs baseline: 2.7342x; 1.0705x over previous
import jax
import jax.numpy as jnp
from jax.experimental import pallas as pl
from jax.experimental.pallas import tpu as pltpu


def _fused_kernel(x_ref, w_ref, bt_ref, biaT_ref, labT_ref, am_ref,
                  ap_ref, op_ref, ag_ref, og_ref,
                  m_ref, lS_ref, lE_ref, la_ref, lo_ref,
                  bits_ref, acc_ref):
    b = pl.program_id(0)
    m = pl.program_id(1)
    nb = pl.num_programs(0)
    nm = pl.num_programs(1)
    blk = x_ref.shape[1]

    @pl.when((b == 0) & (m == 0))
    def _init():
        for i in range(6):
            acc_ref[i] = jnp.float32(0.0)

    lt2 = jnp.dot(x_ref[0], w_ref[...], preferred_element_type=jnp.float32)
    ltT = jnp.transpose(lt2)
    xsc = (ltT + bt_ref[...]) * (1.0 + biaT_ref[0])
    labT = labT_ref[0]
    w = (labT[0:1, :] >= 0).astype(jnp.float32)
    y = labT.astype(jnp.float32)
    elem = jnp.maximum(xsc, 0.0) - xsc * y + jnp.log1p(jnp.exp(-jnp.abs(xsc)))
    welem = w * elem
    acc_ref[0] += jnp.sum(welem[0:1, :])
    acc_ref[1] += jnp.sum(welem[1:2, :])
    pred = jax.nn.sigmoid(xsc) * w
    bits_ref[:, pl.ds(m * blk, blk)] = jax.lax.bitcast_convert_type(
        pred, jnp.int32)

    @pl.when(m == 0)
    def _ce():
        def ce_acc(x, tgt, i0):
            mx = jnp.max(x, axis=0, keepdims=True)
            lse = mx + jnp.log(jnp.sum(jnp.exp(x - mx), axis=0, keepdims=True))
            valid = tgt != -1
            vf = valid.astype(jnp.float32)
            st = jnp.where(valid, tgt, 0)
            oh = (jax.lax.broadcasted_iota(jnp.int32, x.shape, 0) == st
                  ).astype(jnp.float32)
            xt = jnp.sum(x * oh, axis=0, keepdims=True)
            acc_ref[i0] += jnp.sum((lse - xt) * vf)
            acc_ref[i0 + 1] += jnp.sum(vf)

        ce_acc(ap_ref[0], ag_ref[0], 2)
        ce_acc(op_ref[0], og_ref[0], 4)

    @pl.when(m == nm - 1)
    def _prune():
        mask_len = jnp.sum(am_ref[0]) - 2
        length = (mask_len.astype(jnp.float32) * 0.3).astype(jnp.int32)
        length = jnp.maximum(length, 5)
        length = jnp.minimum(length, mask_len * mask_len)
        bits = bits_ref[...]

        def body(_, lh):
            lo, hi = lh
            mid = jax.lax.div(lo + hi, jnp.int32(2))
            cnt = jnp.sum((bits >= mid).astype(jnp.int32), axis=1,
                          keepdims=True)
            ok = cnt >= length
            return jnp.where(ok, mid, lo), jnp.where(ok, hi, mid)

        lo, _ = jax.lax.fori_loop(
            0, 31, body,
            (jnp.zeros((2, 1), jnp.int32),
             jnp.full((2, 1), 0x3F800001, jnp.int32)))
        m_ref[0] = (bits >= lo).astype(jnp.int32)

    @pl.when((b == nb - 1) & (m == nm - 1))
    def _final():
        nelem = jnp.float32(nb * nm * blk)
        lS_ref[...] = jnp.reshape(acc_ref[0] / nelem, (1, 1))
        lE_ref[...] = jnp.reshape(acc_ref[1] / nelem, (1, 1))
        la_ref[...] = jnp.reshape(
            0.1 * acc_ref[2] / jnp.maximum(acc_ref[3], 1.0), (1, 1))
        lo_ref[...] = jnp.reshape(
            0.1 * acc_ref[4] / jnp.maximum(acc_ref[5], 1.0), (1, 1))


def kernel(table, attention_mask, table_labels_S, table_labels_E,
           aspect_pred_tags, opinion_pred_tags, aspect_golde_tags,
           opinion_golde_tags, biaffine_edge_S, biaffine_edge_E,
           W_S, b_S, W_E, b_E):
    B, Lq, Lk, D = table.shape
    LL = Lq * Lk
    BLK = 8192
    NM = LL // BLK

    x = table.reshape(B, LL, D)
    W2 = jnp.concatenate([W_S, W_E], axis=1)
    b2T = jnp.concatenate([b_S, b_E]).reshape(2, 1)
    biaT = jnp.stack([biaffine_edge_S.reshape(B, LL),
                      biaffine_edge_E.reshape(B, LL)], axis=1)
    labT = jnp.stack([table_labels_S.reshape(B, LL),
                      table_labels_E.reshape(B, LL)], axis=1)
    am3 = attention_mask.reshape(B, 1, Lq)
    ap_t = jnp.transpose(aspect_pred_tags, (0, 2, 1))
    op_t = jnp.transpose(opinion_pred_tags, (0, 2, 1))
    ag3 = aspect_golde_tags.reshape(B, 1, Lq)
    og3 = opinion_golde_tags.reshape(B, 1, Lq)
    C = ap_t.shape[1]

    outs = pl.pallas_call(
        _fused_kernel,
        grid=(B, NM),
        in_specs=[
            pl.BlockSpec((1, BLK, D), lambda b, m: (b, m, 0)),
            pl.BlockSpec((D, 2), lambda b, m: (0, 0)),
            pl.BlockSpec((2, 1), lambda b, m: (0, 0)),
            pl.BlockSpec((1, 2, BLK), lambda b, m: (b, 0, m)),
            pl.BlockSpec((1, 2, BLK), lambda b, m: (b, 0, m)),
            pl.BlockSpec((1, 1, Lq), lambda b, m: (b, 0, 0)),
            pl.BlockSpec((1, C, Lq), lambda b, m: (b, 0, 0)),
            pl.BlockSpec((1, C, Lq), lambda b, m: (b, 0, 0)),
            pl.BlockSpec((1, 1, Lq), lambda b, m: (b, 0, 0)),
            pl.BlockSpec((1, 1, Lq), lambda b, m: (b, 0, 0)),
        ],
        out_specs=[
            pl.BlockSpec((1, 2, LL), lambda b, m: (b, 0, 0)),
            pl.BlockSpec((1, 1), lambda b, m: (0, 0)),
            pl.BlockSpec((1, 1), lambda b, m: (0, 0)),
            pl.BlockSpec((1, 1), lambda b, m: (0, 0)),
            pl.BlockSpec((1, 1), lambda b, m: (0, 0)),
        ],
        out_shape=[
            jax.ShapeDtypeStruct((B, 2, LL), jnp.int32),
            jax.ShapeDtypeStruct((1, 1), jnp.float32),
            jax.ShapeDtypeStruct((1, 1), jnp.float32),
            jax.ShapeDtypeStruct((1, 1), jnp.float32),
            jax.ShapeDtypeStruct((1, 1), jnp.float32),
        ],
        scratch_shapes=[
            pltpu.VMEM((2, LL), jnp.int32),
            pltpu.SMEM((8,), jnp.float32),
        ],
    )(x, W2, b2T, biaT, labT, am3, ap_t, op_t, ag3, og3)

    masks, lS, lE, la, lo = outs
    mS = masks[:, 0, :].reshape(B, Lq, Lk).astype(jnp.bool_)
    mE = masks[:, 1, :].reshape(B, Lq, Lk).astype(jnp.bool_)
    return (lS.reshape(()), lE.reshape(()), la.reshape(()), lo.reshape(()),
            mS, mE)
